# baseline (speedup 1.0000x reference)
import functools

import jax
import jax.numpy as jnp
from jax import lax
from jax.experimental import pallas as pl
from jax.experimental.pallas import tpu as pltpu
from jax.experimental.pallas import tpu_sc as plsc

N = 10000
D = 128
DE = 4
NC = 2
NS = 16
NW = NC * NS
B = 128
NCHUNK = D // 16
NBUF = 2
IBUF = 4

ROWS_PER_TILE = 640
N_PAD = NS * ROWS_PER_TILE
OCHUNK = 128


def _sc_edge_kernel(bpw0: int, bpw1: int):
    mesh = plsc.VectorSubcoreMesh(
        core_axis_name="c", subcore_axis_name="s", num_cores=NC, num_subcores=NS
    )

    @functools.partial(
        pl.kernel,
        out_type=jax.ShapeDtypeStruct((NC * N_PAD, D), jnp.float32),
        mesh=mesh,
        scratch_types=[
            pltpu.VMEM((IBUF, 2, B), jnp.int32),
            pltpu.VMEM((IBUF, DE, B), jnp.float32),
            pltpu.VMEM((NBUF, B, D), jnp.float32),
            pltpu.VMEM((DE, D), jnp.float32),
            pltpu.VMEM((D,), jnp.float32),
            pltpu.VMEM_SHARED((N_PAD, D), jnp.float32),
            pltpu.SemaphoreType.DMA((IBUF,)),
            pltpu.SemaphoreType.DMA((IBUF,)),
            pltpu.SemaphoreType.DMA((NBUF,)),
            pltpu.SemaphoreType.DMA((NBUF,)),
        ],
    )
    def k(x_hbm, sd_hbm, attr_hbm, we_hbm, be_hbm, out_hbm,
          sd, av_ring, gbuf, we_v, be_v, accum, isem, asem, gsem, ssem):
        cid = lax.axis_index("c")
        tid = lax.axis_index("s")
        bpw = jnp.where(cid == 0, bpw0, bpw1)
        wbase = cid * (NS * bpw0) + tid * bpw

        pltpu.sync_copy(we_hbm, we_v)
        pltpu.sync_copy(be_hbm, be_v)

        zero16 = jnp.zeros((16,), jnp.float32)

        def zrow(i, _):
            for c in range(NCHUNK):
                gbuf[0, i, pl.ds(c * 16, 16)] = zero16
            return 0

        lax.fori_loop(0, OCHUNK, zrow, 0)
        for j in range(ROWS_PER_TILE // OCHUNK):
            pltpu.sync_copy(
                gbuf.at[0, pl.ds(0, OCHUNK)],
                accum.at[pl.ds(tid * ROWS_PER_TILE + j * OCHUNK, OCHUNK)])
        plsc.subcore_barrier()

        wv = [[we_v[kk, pl.ds(c * 16, 16)] for kk in range(DE)]
              for c in range(NCHUNK)]
        bv = [be_v[pl.ds(c * 16, 16)] for c in range(NCHUNK)]

        def load_idx(i):
            s = lax.rem(i, IBUF)
            pltpu.async_copy(sd_hbm.at[wbase + i], sd.at[s], isem.at[s])
            pltpu.async_copy(attr_hbm.at[wbase + i], av_ring.at[s],
                             asem.at[s])

        def load_idx_wait(i):
            s = lax.rem(i, IBUF)
            pltpu.make_async_copy(sd_hbm.at[wbase + i], sd.at[s],
                                  isem.at[s]).wait()
            pltpu.make_async_copy(attr_hbm.at[wbase + i], av_ring.at[s],
                                  asem.at[s]).wait()

        def gather(i):
            s, p = lax.rem(i, IBUF), lax.rem(i, NBUF)
            pltpu.async_copy(x_hbm.at[sd.at[s, 0]], gbuf.at[p], gsem.at[p])

        def gather_wait(i):
            s, p = lax.rem(i, IBUF), lax.rem(i, NBUF)
            pltpu.make_async_copy(x_hbm.at[sd.at[s, 0]], gbuf.at[p],
                                  gsem.at[p]).wait()

        def scatter(i):
            s, p = lax.rem(i, IBUF), lax.rem(i, NBUF)
            idx = plsc.Indices(sd.at[s, 1], ignored_value=-1)
            pltpu.async_copy(gbuf.at[p], accum.at[idx], ssem.at[p], add=True)

        def scatter_wait(i):
            s, p = lax.rem(i, IBUF), lax.rem(i, NBUF)
            idx = plsc.Indices(sd.at[s, 1], ignored_value=-1)
            pltpu.make_async_copy(gbuf.at[p], accum.at[idx], ssem.at[p]).wait()

        def compute(i):
            s, p = lax.rem(i, IBUF), lax.rem(i, NBUF)

            @plsc.parallel_loop(0, B // 16, 1, unroll=1)
            def group_body(g):
                av = [av_ring[s, kk, pl.ds(g * 16, 16)] for kk in range(DE)]
                for j in range(16):
                    e = g * 16 + j
                    a = [av[kk][j] for kk in range(DE)]
                    gvs = [gbuf[p, e, pl.ds(c * 16, 16)]
                           for c in range(NCHUNK)]
                    ms = []
                    for c in range(NCHUNK):
                        t1 = a[0] * wv[c][0] + a[1] * wv[c][1]
                        t2 = a[2] * wv[c][2] + a[3] * wv[c][3]
                        ms.append(jnp.maximum((gvs[c] + bv[c]) + (t1 + t2),
                                              0.0))
                    for c in range(NCHUNK):
                        gbuf[p, e, pl.ds(c * 16, 16)] = ms[c]

        load_idx(0)
        load_idx(1)
        load_idx_wait(0)
        gather(0)

        def step(i, _):
            @pl.when(i >= 1)
            def _():
                scatter_wait(i - 1)

            @pl.when(i + 2 < bpw)
            def _():
                load_idx(i + 2)

            @pl.when(i + 1 < bpw)
            def _():
                load_idx_wait(i + 1)
                gather(i + 1)

            gather_wait(i)
            compute(i)
            scatter(i)
            return 0

        lax.fori_loop(0, bpw, step, 0)
        scatter_wait(bpw - 1)
        plsc.subcore_barrier()

        for j in range(ROWS_PER_TILE // OCHUNK):
            off = tid * ROWS_PER_TILE + j * OCHUNK
            pltpu.sync_copy(accum.at[pl.ds(off, OCHUNK)],
                            gbuf.at[0, pl.ds(0, OCHUNK)])
            pltpu.sync_copy(gbuf.at[0, pl.ds(0, OCHUNK)],
                            out_hbm.at[pl.ds(cid * N_PAD + off, OCHUNK)])

    return k


def _mlp_kernel(p0_ref, p1_ref, x_ref, w1_ref, b1_ref, w2_ref, b2_ref, o_ref):
    h = p0_ref[...] + p1_ref[...] + x_ref[...]
    h1 = jnp.maximum(
        jnp.dot(h, w1_ref[...], preferred_element_type=jnp.float32)
        + b1_ref[...], 0.0)
    o_ref[...] = (jnp.dot(h1, w2_ref[...], preferred_element_type=jnp.float32)
                  + b2_ref[...])


BPW0 = 84
BPW1 = 73


def kernel(x, edge_index, edge_attr, We, be, W1, b1, W2, b2):
    e = edge_index.shape[1]
    bpw_sum = -(-e // (NS * B))
    bpw0 = min(BPW0, bpw_sum)
    bpw1 = bpw_sum - bpw0
    e_pad = NS * bpw_sum * B
    src = edge_index[0].astype(jnp.int32)
    dst = edge_index[1].astype(jnp.int32)
    pad = e_pad - e
    src = jnp.pad(src, (0, pad))
    dst = jnp.pad(dst, (0, pad), constant_values=-1)
    attr = jnp.pad(edge_attr.astype(jnp.float32), ((0, pad), (0, 0)))
    nb = e_pad // B
    sd3 = jnp.stack([src.reshape(nb, B), dst.reshape(nb, B)], axis=1)
    attr3 = attr.reshape(nb, B, DE).transpose(0, 2, 1)

    parts = _sc_edge_kernel(bpw0, bpw1)(x, sd3, attr3, We, be)

    p0 = parts[:N]
    p1 = parts[N_PAD:N_PAD + N]

    blk = 1000
    out = pl.pallas_call(
        _mlp_kernel,
        grid=(N // blk,),
        in_specs=[
            pl.BlockSpec((blk, D), lambda i: (i, 0)),
            pl.BlockSpec((blk, D), lambda i: (i, 0)),
            pl.BlockSpec((blk, D), lambda i: (i, 0)),
            pl.BlockSpec((D, D), lambda i: (0, 0)),
            pl.BlockSpec((1, D), lambda i: (0, 0)),
            pl.BlockSpec((D, D), lambda i: (0, 0)),
            pl.BlockSpec((1, D), lambda i: (0, 0)),
        ],
        out_specs=pl.BlockSpec((blk, D), lambda i: (i, 0)),
        out_shape=jax.ShapeDtypeStruct((N, D), jnp.float32),
    )(p0, p1, x, W1, b1.reshape(1, D), W2, b2.reshape(1, D))
    return out


# speedup vs baseline: 1.0051x; 1.0051x over previous
import functools

import jax
import jax.numpy as jnp
from jax import lax
from jax.experimental import pallas as pl
from jax.experimental.pallas import tpu as pltpu
from jax.experimental.pallas import tpu_sc as plsc

N = 10000
D = 128
DE = 4
NC = 2
NS = 16
NW = NC * NS
B = 128
NCHUNK = D // 16
NBUF = 2
IBUF = 4

ROWS_PER_TILE = 640
N_PAD = NS * ROWS_PER_TILE
OCHUNK = 128

BPW0 = 88
BPW1 = 69


def _sc_edge_kernel(bpw0: int, bpw1: int):
    mesh = plsc.VectorSubcoreMesh(
        core_axis_name="c", subcore_axis_name="s", num_cores=NC, num_subcores=NS
    )

    @functools.partial(
        pl.kernel,
        out_type=jax.ShapeDtypeStruct((NC * N_PAD, D), jnp.float32),
        mesh=mesh,
        scratch_types=[
            pltpu.VMEM((IBUF, 2, B), jnp.int32),
            pltpu.VMEM((IBUF, DE, B), jnp.float32),
            pltpu.VMEM((NBUF, B, D), jnp.float32),
            pltpu.VMEM((DE, D), jnp.float32),
            pltpu.VMEM((D,), jnp.float32),
            pltpu.VMEM_SHARED((N_PAD, D), jnp.float32),
            pltpu.SemaphoreType.DMA((IBUF,)),
            pltpu.SemaphoreType.DMA((IBUF,)),
            pltpu.SemaphoreType.DMA((NBUF,)),
            pltpu.SemaphoreType.DMA,
        ],
    )
    def k(x_hbm, sd_hbm, attr_hbm, we_hbm, be_hbm, out_hbm,
          sd, av_ring, gbuf, we_v, be_v, accum, isem, asem, gsem, ssem):
        cid = lax.axis_index("c")
        tid = lax.axis_index("s")
        bpw = jnp.where(cid == 0, bpw0, bpw1)
        wbase = cid * (NS * bpw0) + tid * bpw

        pltpu.sync_copy(we_hbm, we_v)
        pltpu.sync_copy(be_hbm, be_v)

        zero16 = jnp.zeros((16,), jnp.float32)

        def zrow(i, _):
            for c in range(NCHUNK):
                gbuf[0, i, pl.ds(c * 16, 16)] = zero16
            return 0

        lax.fori_loop(0, OCHUNK, zrow, 0)
        for j in range(ROWS_PER_TILE // OCHUNK):
            pltpu.sync_copy(
                gbuf.at[0],
                accum.at[pl.ds(tid * ROWS_PER_TILE + j * OCHUNK, OCHUNK)])
        plsc.subcore_barrier()

        wv = [[we_v[kk, pl.ds(c * 16, 16)] for kk in range(DE)]
              for c in range(NCHUNK)]
        bv = [be_v[pl.ds(c * 16, 16)] for c in range(NCHUNK)]

        def load_idx(i):
            s = lax.rem(i, IBUF)
            pltpu.async_copy(sd_hbm.at[wbase + i], sd.at[s], isem.at[s])
            pltpu.async_copy(attr_hbm.at[wbase + i], av_ring.at[s],
                             asem.at[s])

        def load_idx_wait(i):
            s = lax.rem(i, IBUF)
            pltpu.make_async_copy(sd_hbm.at[wbase + i], sd.at[s],
                                  isem.at[s]).wait()
            pltpu.make_async_copy(attr_hbm.at[wbase + i], av_ring.at[s],
                                  asem.at[s]).wait()

        def gather(i):
            s, p = lax.rem(i, IBUF), lax.rem(i, NBUF)
            pltpu.async_copy(x_hbm.at[sd.at[s, 0, pl.ds(0, B // 2)]],
                             gbuf.at[p, pl.ds(0, B // 2)], gsem.at[p])
            pltpu.async_copy(x_hbm.at[sd.at[s, 0, pl.ds(B // 2, B // 2)]],
                             gbuf.at[p, pl.ds(B // 2, B // 2)], gsem.at[p])

        def gather_wait(i):
            s, p = lax.rem(i, IBUF), lax.rem(i, NBUF)
            pltpu.make_async_copy(x_hbm.at[sd.at[s, 0, pl.ds(0, B // 2)]],
                                  gbuf.at[p, pl.ds(0, B // 2)],
                                  gsem.at[p]).wait()
            pltpu.make_async_copy(x_hbm.at[sd.at[s, 0, pl.ds(B // 2, B // 2)]],
                                  gbuf.at[p, pl.ds(B // 2, B // 2)],
                                  gsem.at[p]).wait()

        def scatter(i):
            s, p = lax.rem(i, IBUF), lax.rem(i, NBUF)
            idx = plsc.Indices(sd.at[s, 1], ignored_value=-1)
            pltpu.async_copy(gbuf.at[p], accum.at[idx], ssem, add=True)

        def scatter_wait(i):
            s, p = lax.rem(i, IBUF), lax.rem(i, NBUF)
            idx = plsc.Indices(sd.at[s, 1], ignored_value=-1)
            pltpu.make_async_copy(gbuf.at[p], accum.at[idx], ssem).wait()

        def compute(i):
            s, p = lax.rem(i, IBUF), lax.rem(i, NBUF)

            @plsc.parallel_loop(0, B // 16, 1, unroll=1)
            def group_body(g):
                av = [av_ring[s, kk, pl.ds(g * 16, 16)] for kk in range(DE)]
                for j in range(16):
                    e = g * 16 + j
                    a = [av[kk][j] for kk in range(DE)]
                    gvs = [gbuf[p, e, pl.ds(c * 16, 16)]
                           for c in range(NCHUNK)]
                    ms = []
                    for c in range(NCHUNK):
                        t1 = a[0] * wv[c][0] + a[1] * wv[c][1]
                        t2 = a[2] * wv[c][2] + a[3] * wv[c][3]
                        ms.append(jnp.maximum((gvs[c] + bv[c]) + (t1 + t2),
                                              0.0))
                    for c in range(NCHUNK):
                        gbuf[p, e, pl.ds(c * 16, 16)] = ms[c]

        load_idx(0)
        load_idx(1)
        load_idx_wait(0)
        gather(0)

        def step(i, _):
            @pl.when(i >= 1)
            def _():
                scatter_wait(i - 1)

            @pl.when(i + 2 < bpw)
            def _():
                load_idx(i + 2)

            @pl.when(i + 1 < bpw)
            def _():
                load_idx_wait(i + 1)
                gather(i + 1)

            gather_wait(i)
            compute(i)
            scatter(i)
            return 0

        lax.fori_loop(0, bpw, step, 0)
        scatter_wait(bpw - 1)
        plsc.subcore_barrier()

        for j in range(ROWS_PER_TILE // OCHUNK):
            off = tid * ROWS_PER_TILE + j * OCHUNK
            pltpu.sync_copy(accum.at[pl.ds(off, OCHUNK)], gbuf.at[0])
            pltpu.sync_copy(gbuf.at[0],
                            out_hbm.at[pl.ds(cid * N_PAD + off, OCHUNK)])

    return k


def _mlp_kernel(p0_ref, p1_ref, x_ref, w1_ref, b1_ref, w2_ref, b2_ref, o_ref):
    h = p0_ref[...] + p1_ref[...] + x_ref[...]
    h1 = jnp.maximum(
        jnp.dot(h, w1_ref[...], preferred_element_type=jnp.float32)
        + b1_ref[...], 0.0)
    o_ref[...] = (jnp.dot(h1, w2_ref[...], preferred_element_type=jnp.float32)
                  + b2_ref[...])


def kernel(x, edge_index, edge_attr, We, be, W1, b1, W2, b2):
    e = edge_index.shape[1]
    bpw_sum = -(-e // (NS * B))
    bpw0 = min(BPW0, bpw_sum)
    bpw1 = bpw_sum - bpw0
    e_pad = NS * bpw_sum * B
    src = edge_index[0].astype(jnp.int32)
    dst = edge_index[1].astype(jnp.int32)
    pad = e_pad - e
    src = jnp.pad(src, (0, pad))
    dst = jnp.pad(dst, (0, pad), constant_values=-1)
    attr = jnp.pad(edge_attr.astype(jnp.float32), ((0, pad), (0, 0)))
    nb = e_pad // B
    sd3 = jnp.stack([src.reshape(nb, B), dst.reshape(nb, B)], axis=1)
    attr3 = attr.reshape(nb, B, DE).transpose(0, 2, 1)
    parts = _sc_edge_kernel(bpw0, bpw1)(x, sd3, attr3, We, be)

    p0 = parts[:N]
    p1 = parts[N_PAD:N_PAD + N]

    blk = 1000
    out = pl.pallas_call(
        _mlp_kernel,
        grid=(N // blk,),
        in_specs=[
            pl.BlockSpec((blk, D), lambda i: (i, 0)),
            pl.BlockSpec((blk, D), lambda i: (i, 0)),
            pl.BlockSpec((blk, D), lambda i: (i, 0)),
            pl.BlockSpec((D, D), lambda i: (0, 0)),
            pl.BlockSpec((1, D), lambda i: (0, 0)),
            pl.BlockSpec((D, D), lambda i: (0, 0)),
            pl.BlockSpec((1, D), lambda i: (0, 0)),
        ],
        out_specs=pl.BlockSpec((blk, D), lambda i: (i, 0)),
        out_shape=jax.ShapeDtypeStruct((N, D), jnp.float32),
    )(p0, p1, x, W1, b1.reshape(1, D), W2, b2.reshape(1, D))
    return out


# speedup vs baseline: 1.0524x; 1.0470x over previous
import functools

import jax
import jax.numpy as jnp
from jax import lax
from jax.experimental import pallas as pl
from jax.experimental.pallas import tpu as pltpu
from jax.experimental.pallas import tpu_sc as plsc

N = 10000
D = 128
DE = 4
NC = 2
NS = 16
NW = NC * NS
B = 128
NCHUNK = D // 16
NBUF = 2
IBUF = 4

ROWS_PER_TILE = 640
N_PAD = NS * ROWS_PER_TILE
OCHUNK = 128

BPW0 = 88
BPW1 = 69


def _sc_edge_kernel(bpw0: int, bpw1: int):
    mesh = plsc.VectorSubcoreMesh(
        core_axis_name="c", subcore_axis_name="s", num_cores=NC, num_subcores=NS
    )

    @functools.partial(
        pl.kernel,
        out_type=jax.ShapeDtypeStruct((NC * N_PAD, D), jnp.float32),
        mesh=mesh,
        scratch_types=[
            pltpu.VMEM((IBUF, 4, B // 2), jnp.int32),
            pltpu.VMEM((IBUF, DE, B), jnp.float32),
            pltpu.VMEM((NBUF, B, D), jnp.float32),
            pltpu.VMEM((DE, D), jnp.float32),
            pltpu.VMEM((D,), jnp.float32),
            pltpu.VMEM_SHARED((N_PAD, D), jnp.float32),
            pltpu.SemaphoreType.DMA((IBUF,)),
            pltpu.SemaphoreType.DMA((IBUF,)),
            pltpu.SemaphoreType.DMA((NBUF,)),
            pltpu.SemaphoreType.DMA,
        ],
    )
    def k(x_hbm, sd_hbm, attr_hbm, we_hbm, be_hbm, out_hbm,
          sd, av_ring, gbuf, we_v, be_v, accum, isem, asem, gsem, ssem):
        cid = lax.axis_index("c")
        tid = lax.axis_index("s")
        bpw = jnp.where(cid == 0, bpw0, bpw1)
        wbase = cid * (NS * bpw0) + tid * bpw

        pltpu.sync_copy(we_hbm, we_v)
        pltpu.sync_copy(be_hbm, be_v)

        zero16 = jnp.zeros((16,), jnp.float32)

        def zrow(i, _):
            for c in range(NCHUNK):
                gbuf[0, i, pl.ds(c * 16, 16)] = zero16
            return 0

        lax.fori_loop(0, OCHUNK, zrow, 0)
        for j in range(ROWS_PER_TILE // OCHUNK):
            pltpu.sync_copy(
                gbuf.at[0],
                accum.at[pl.ds(tid * ROWS_PER_TILE + j * OCHUNK, OCHUNK)])
        plsc.subcore_barrier()

        wv = [[we_v[kk, pl.ds(c * 16, 16)] for kk in range(DE)]
              for c in range(NCHUNK)]
        bv = [be_v[pl.ds(c * 16, 16)] for c in range(NCHUNK)]

        def load_idx(i):
            s = lax.rem(i, IBUF)
            pltpu.async_copy(sd_hbm.at[wbase + i], sd.at[s], isem.at[s])
            pltpu.async_copy(attr_hbm.at[wbase + i], av_ring.at[s],
                             asem.at[s])

        def load_idx_wait(i):
            s = lax.rem(i, IBUF)
            pltpu.make_async_copy(sd_hbm.at[wbase + i], sd.at[s],
                                  isem.at[s]).wait()
            pltpu.make_async_copy(attr_hbm.at[wbase + i], av_ring.at[s],
                                  asem.at[s]).wait()

        def gather(i):
            s, p = lax.rem(i, IBUF), lax.rem(i, NBUF)
            pltpu.async_copy(x_hbm.at[sd.at[s, 0]],
                             gbuf.at[p, pl.ds(0, B // 2)], gsem.at[p])
            pltpu.async_copy(x_hbm.at[sd.at[s, 1]],
                             gbuf.at[p, pl.ds(B // 2, B // 2)], gsem.at[p])

        def gather_wait(i):
            s, p = lax.rem(i, IBUF), lax.rem(i, NBUF)
            pltpu.make_async_copy(x_hbm.at[sd.at[s, 0]],
                                  gbuf.at[p, pl.ds(0, B // 2)],
                                  gsem.at[p]).wait()
            pltpu.make_async_copy(x_hbm.at[sd.at[s, 1]],
                                  gbuf.at[p, pl.ds(B // 2, B // 2)],
                                  gsem.at[p]).wait()

        def scatter_half(i, h):
            s, p = lax.rem(i, IBUF), lax.rem(i, NBUF)
            idx = plsc.Indices(sd.at[s, 2 + h], ignored_value=-1)
            pltpu.async_copy(gbuf.at[p, pl.ds(h * (B // 2), B // 2)],
                             accum.at[idx], ssem, add=True)

        def scatter_wait(i):
            s, p = lax.rem(i, IBUF), lax.rem(i, NBUF)
            for h in range(2):
                idx = plsc.Indices(sd.at[s, 2 + h], ignored_value=-1)
                pltpu.make_async_copy(
                    gbuf.at[p, pl.ds(h * (B // 2), B // 2)],
                    accum.at[idx], ssem).wait()

        def compute_half(i, h):
            s, p = lax.rem(i, IBUF), lax.rem(i, NBUF)
            gh = (B // 32) * h

            @plsc.parallel_loop(gh, gh + B // 32, 1, unroll=1)
            def group_body(g):
                av = [av_ring[s, kk, pl.ds(g * 16, 16)] for kk in range(DE)]
                for j in range(16):
                    e = g * 16 + j
                    a = [av[kk][j] for kk in range(DE)]
                    gvs = [gbuf[p, e, pl.ds(c * 16, 16)]
                           for c in range(NCHUNK)]
                    ms = []
                    for c in range(NCHUNK):
                        t1 = a[0] * wv[c][0] + a[1] * wv[c][1]
                        t2 = a[2] * wv[c][2] + a[3] * wv[c][3]
                        ms.append(jnp.maximum((gvs[c] + bv[c]) + (t1 + t2),
                                              0.0))
                    for c in range(NCHUNK):
                        gbuf[p, e, pl.ds(c * 16, 16)] = ms[c]

        load_idx(0)
        load_idx(1)
        load_idx_wait(0)
        gather(0)

        def step(i, _):
            @pl.when(i >= 1)
            def _():
                scatter_wait(i - 1)

            @pl.when(i + 2 < bpw)
            def _():
                load_idx(i + 2)

            @pl.when(i + 1 < bpw)
            def _():
                load_idx_wait(i + 1)
                gather(i + 1)

            gather_wait(i)
            compute_half(i, 0)
            scatter_half(i, 0)
            compute_half(i, 1)
            scatter_half(i, 1)
            return 0

        lax.fori_loop(0, bpw, step, 0)
        scatter_wait(bpw - 1)
        plsc.subcore_barrier()

        for j in range(ROWS_PER_TILE // OCHUNK):
            off = tid * ROWS_PER_TILE + j * OCHUNK
            pltpu.async_copy(accum.at[pl.ds(off, OCHUNK)],
                             out_hbm.at[pl.ds(cid * N_PAD + off, OCHUNK)],
                             gsem.at[0])
        for j in range(ROWS_PER_TILE // OCHUNK):
            off = tid * ROWS_PER_TILE + j * OCHUNK
            pltpu.make_async_copy(
                accum.at[pl.ds(off, OCHUNK)],
                out_hbm.at[pl.ds(cid * N_PAD + off, OCHUNK)],
                gsem.at[0]).wait()

    return k


def _mlp_kernel(p0_ref, p1_ref, x_ref, w1_ref, b1_ref, w2_ref, b2_ref, o_ref):
    h = p0_ref[...] + p1_ref[...] + x_ref[...]
    h1 = jnp.maximum(
        jnp.dot(h, w1_ref[...], preferred_element_type=jnp.float32)
        + b1_ref[...], 0.0)
    o_ref[...] = (jnp.dot(h1, w2_ref[...], preferred_element_type=jnp.float32)
                  + b2_ref[...])


def kernel(x, edge_index, edge_attr, We, be, W1, b1, W2, b2):
    e = edge_index.shape[1]
    bpw_sum = -(-e // (NS * B))
    bpw0 = min(BPW0, bpw_sum)
    bpw1 = bpw_sum - bpw0
    e_pad = NS * bpw_sum * B
    src = edge_index[0].astype(jnp.int32)
    dst = edge_index[1].astype(jnp.int32)
    pad = e_pad - e
    src = jnp.pad(src, (0, pad))
    dst = jnp.pad(dst, (0, pad), constant_values=-1)
    attr = jnp.pad(edge_attr.astype(jnp.float32), ((0, pad), (0, 0)))
    nb = e_pad // B
    sd3 = jnp.concatenate(
        [src.reshape(nb, 2, B // 2), dst.reshape(nb, 2, B // 2)], axis=1)
    attr3 = attr.reshape(nb, B, DE).transpose(0, 2, 1)
    parts = _sc_edge_kernel(bpw0, bpw1)(x, sd3, attr3, We, be)

    p0 = parts[:N]
    p1 = parts[N_PAD:N_PAD + N]

    blk = 1000
    out = pl.pallas_call(
        _mlp_kernel,
        grid=(N // blk,),
        in_specs=[
            pl.BlockSpec((blk, D), lambda i: (i, 0)),
            pl.BlockSpec((blk, D), lambda i: (i, 0)),
            pl.BlockSpec((blk, D), lambda i: (i, 0)),
            pl.BlockSpec((D, D), lambda i: (0, 0)),
            pl.BlockSpec((1, D), lambda i: (0, 0)),
            pl.BlockSpec((D, D), lambda i: (0, 0)),
            pl.BlockSpec((1, D), lambda i: (0, 0)),
        ],
        out_specs=pl.BlockSpec((blk, D), lambda i: (i, 0)),
        out_shape=jax.ShapeDtypeStruct((N, D), jnp.float32),
    )(p0, p1, x, W1, b1.reshape(1, D), W2, b2.reshape(1, D))
    return out


# speedup vs baseline: 1.0771x; 1.0234x over previous
import functools

import jax
import jax.numpy as jnp
from jax import lax
from jax.experimental import pallas as pl
from jax.experimental.pallas import tpu as pltpu
from jax.experimental.pallas import tpu_sc as plsc

N = 10000
D = 128
DE = 4
NC = 2
NS = 16
NW = NC * NS
B = 128
NCHUNK = D // 16
NBUF = 2
IBUF = 4

ROWS_PER_TILE = 640
N_PAD = NS * ROWS_PER_TILE
OCHUNK = 128

BPW0 = 88
BPW1 = 69


def _sc_edge_kernel(bpw0: int, bpw1: int):
    mesh = plsc.VectorSubcoreMesh(
        core_axis_name="c", subcore_axis_name="s", num_cores=NC, num_subcores=NS
    )

    @functools.partial(
        pl.kernel,
        out_type=jax.ShapeDtypeStruct((NC * N, D), jnp.float32),
        mesh=mesh,
        scratch_types=[
            pltpu.VMEM((IBUF, 4, B // 2), jnp.int32),
            pltpu.VMEM((IBUF, DE, B), jnp.float32),
            pltpu.VMEM((NBUF, B, D), jnp.float32),
            pltpu.VMEM((DE, D), jnp.float32),
            pltpu.VMEM((D,), jnp.float32),
            pltpu.VMEM_SHARED((N_PAD, D), jnp.float32),
            pltpu.SemaphoreType.DMA((IBUF,)),
            pltpu.SemaphoreType.DMA((IBUF,)),
            pltpu.SemaphoreType.DMA((NBUF,)),
            pltpu.SemaphoreType.DMA,
        ],
    )
    def k(x_hbm, sd_hbm, attr_hbm, we_hbm, be_hbm, out_hbm,
          sd, av_ring, gbuf, we_v, be_v, accum, isem, asem, gsem, ssem):
        cid = lax.axis_index("c")
        tid = lax.axis_index("s")
        bpw = jnp.where(cid == 0, bpw0, bpw1)
        wbase = cid * (NS * bpw0) + tid * bpw

        pltpu.sync_copy(we_hbm, we_v)
        pltpu.sync_copy(be_hbm, be_v)

        zero16 = jnp.zeros((16,), jnp.float32)

        def zrow(i, _):
            for c in range(NCHUNK):
                gbuf[0, i, pl.ds(c * 16, 16)] = zero16
            return 0

        lax.fori_loop(0, OCHUNK, zrow, 0)
        for j in range(ROWS_PER_TILE // OCHUNK):
            pltpu.sync_copy(
                gbuf.at[0],
                accum.at[pl.ds(tid * ROWS_PER_TILE + j * OCHUNK, OCHUNK)])
        plsc.subcore_barrier()

        wv = [[we_v[kk, pl.ds(c * 16, 16)] for kk in range(DE)]
              for c in range(NCHUNK)]
        bv = [be_v[pl.ds(c * 16, 16)] for c in range(NCHUNK)]

        def load_idx(i):
            s = lax.rem(i, IBUF)
            pltpu.async_copy(sd_hbm.at[wbase + i], sd.at[s], isem.at[s])
            pltpu.async_copy(attr_hbm.at[wbase + i], av_ring.at[s],
                             asem.at[s])

        def load_idx_wait(i):
            s = lax.rem(i, IBUF)
            pltpu.make_async_copy(sd_hbm.at[wbase + i], sd.at[s],
                                  isem.at[s]).wait()
            pltpu.make_async_copy(attr_hbm.at[wbase + i], av_ring.at[s],
                                  asem.at[s]).wait()

        def gather(i):
            s, p = lax.rem(i, IBUF), lax.rem(i, NBUF)
            pltpu.async_copy(x_hbm.at[sd.at[s, 0]],
                             gbuf.at[p, pl.ds(0, B // 2)], gsem.at[p])
            pltpu.async_copy(x_hbm.at[sd.at[s, 1]],
                             gbuf.at[p, pl.ds(B // 2, B // 2)], gsem.at[p])

        def gather_wait(i):
            s, p = lax.rem(i, IBUF), lax.rem(i, NBUF)
            pltpu.make_async_copy(x_hbm.at[sd.at[s, 0]],
                                  gbuf.at[p, pl.ds(0, B // 2)],
                                  gsem.at[p]).wait()
            pltpu.make_async_copy(x_hbm.at[sd.at[s, 1]],
                                  gbuf.at[p, pl.ds(B // 2, B // 2)],
                                  gsem.at[p]).wait()

        def scatter_half(i, h):
            s, p = lax.rem(i, IBUF), lax.rem(i, NBUF)
            idx = plsc.Indices(sd.at[s, 2 + h], ignored_value=-1)
            pltpu.async_copy(gbuf.at[p, pl.ds(h * (B // 2), B // 2)],
                             accum.at[idx], ssem, add=True)

        def scatter_wait(i):
            s, p = lax.rem(i, IBUF), lax.rem(i, NBUF)
            for h in range(2):
                idx = plsc.Indices(sd.at[s, 2 + h], ignored_value=-1)
                pltpu.make_async_copy(
                    gbuf.at[p, pl.ds(h * (B // 2), B // 2)],
                    accum.at[idx], ssem).wait()

        def compute_half(i, h):
            s, p = lax.rem(i, IBUF), lax.rem(i, NBUF)
            gh = (B // 32) * h

            @plsc.parallel_loop(gh, gh + B // 32, 1, unroll=1)
            def group_body(g):
                av = [av_ring[s, kk, pl.ds(g * 16, 16)] for kk in range(DE)]
                for j in range(16):
                    e = g * 16 + j
                    a = [av[kk][j] for kk in range(DE)]
                    gvs = [gbuf[p, e, pl.ds(c * 16, 16)]
                           for c in range(NCHUNK)]
                    ms = []
                    for c in range(NCHUNK):
                        t1 = a[0] * wv[c][0] + a[1] * wv[c][1]
                        t2 = a[2] * wv[c][2] + a[3] * wv[c][3]
                        ms.append(jnp.maximum((gvs[c] + bv[c]) + (t1 + t2),
                                              0.0))
                    for c in range(NCHUNK):
                        gbuf[p, e, pl.ds(c * 16, 16)] = ms[c]

        load_idx(0)
        load_idx(1)
        load_idx_wait(0)
        gather(0)

        def step(i, _):
            @pl.when(i >= 1)
            def _():
                scatter_wait(i - 1)

            @pl.when(i + 2 < bpw)
            def _():
                load_idx(i + 2)

            @pl.when(i + 1 < bpw)
            def _():
                load_idx_wait(i + 1)
                gather(i + 1)

            gather_wait(i)
            compute_half(i, 0)
            scatter_half(i, 0)
            compute_half(i, 1)
            scatter_half(i, 1)
            return 0

        lax.fori_loop(0, bpw, step, 0)
        scatter_wait(bpw - 1)
        plsc.subcore_barrier()

        def owrite(go):
            for j in range(ROWS_PER_TILE // OCHUNK):
                off = tid * ROWS_PER_TILE + j * OCHUNK

                @pl.when(off + OCHUNK <= N)
                def _():
                    go(accum.at[pl.ds(off, OCHUNK)],
                       out_hbm.at[pl.ds(cid * N + off, OCHUNK)])

                @pl.when(off == (N // OCHUNK) * OCHUNK)
                def _():
                    t = N - (N // OCHUNK) * OCHUNK
                    go(accum.at[pl.ds(off, t)],
                       out_hbm.at[pl.ds(cid * N + off, t)])

        owrite(lambda a, b: pltpu.async_copy(a, b, gsem.at[0]))
        owrite(lambda a, b: pltpu.make_async_copy(a, b, gsem.at[0]).wait())

    return k


def _mlp_kernel(p0_ref, p1_ref, x_ref, w1_ref, b1_ref, w2_ref, b2_ref, o_ref):
    h = p0_ref[...] + p1_ref[...] + x_ref[...]
    h1 = jnp.maximum(
        jnp.dot(h, w1_ref[...], preferred_element_type=jnp.float32)
        + b1_ref[...], 0.0)
    o_ref[...] = (jnp.dot(h1, w2_ref[...], preferred_element_type=jnp.float32)
                  + b2_ref[...])


def kernel(x, edge_index, edge_attr, We, be, W1, b1, W2, b2):
    e = edge_index.shape[1]
    bpw_sum = -(-e // (NS * B))
    bpw0 = min(BPW0, bpw_sum)
    bpw1 = bpw_sum - bpw0
    e_pad = NS * bpw_sum * B
    src = edge_index[0].astype(jnp.int32)
    dst = edge_index[1].astype(jnp.int32)
    pad = e_pad - e
    src = jnp.pad(src, (0, pad))
    dst = jnp.pad(dst, (0, pad), constant_values=-1)
    attr = jnp.pad(edge_attr.astype(jnp.float32), ((0, pad), (0, 0)))
    nb = e_pad // B
    sd3 = jnp.concatenate(
        [src.reshape(nb, 2, B // 2), dst.reshape(nb, 2, B // 2)], axis=1)
    attr3 = attr.reshape(nb, B, DE).transpose(0, 2, 1)
    parts = _sc_edge_kernel(bpw0, bpw1)(x, sd3, attr3, We, be)

    blk = 1000
    out = pl.pallas_call(
        _mlp_kernel,
        grid=(N // blk,),
        in_specs=[
            pl.BlockSpec((blk, D), lambda i: (i, 0)),
            pl.BlockSpec((blk, D), lambda i: (N // blk + i, 0)),
            pl.BlockSpec((blk, D), lambda i: (i, 0)),
            pl.BlockSpec((D, D), lambda i: (0, 0)),
            pl.BlockSpec((1, D), lambda i: (0, 0)),
            pl.BlockSpec((D, D), lambda i: (0, 0)),
            pl.BlockSpec((1, D), lambda i: (0, 0)),
        ],
        out_specs=pl.BlockSpec((blk, D), lambda i: (i, 0)),
        out_shape=jax.ShapeDtypeStruct((N, D), jnp.float32),
    )(parts, parts, x, W1, b1.reshape(1, D), W2, b2.reshape(1, D))
    return out


# speedup vs baseline: 1.0792x; 1.0020x over previous
import functools

import jax
import jax.numpy as jnp
from jax import lax
from jax.experimental import pallas as pl
from jax.experimental.pallas import tpu as pltpu
from jax.experimental.pallas import tpu_sc as plsc

N = 10000
D = 128
DE = 4
NC = 2
NS = 16
NW = NC * NS
B = 128
NCHUNK = D // 16
NBUF = 2
IBUF = 4

ROWS_PER_TILE = 640
N_PAD = NS * ROWS_PER_TILE
OCHUNK = 128

BPW0 = 88
BPW1 = 69


def _sc_edge_kernel(bpw0: int, bpw1: int):
    mesh = plsc.VectorSubcoreMesh(
        core_axis_name="c", subcore_axis_name="s", num_cores=NC, num_subcores=NS
    )

    @functools.partial(
        pl.kernel,
        out_type=jax.ShapeDtypeStruct((NC * N, D), jnp.float32),
        mesh=mesh,
        scratch_types=[
            pltpu.VMEM((IBUF, 4, B // 2), jnp.int32),
            pltpu.VMEM((IBUF, DE, B), jnp.float32),
            pltpu.VMEM((NBUF, B, D), jnp.float32),
            pltpu.VMEM((DE, D), jnp.float32),
            pltpu.VMEM((D,), jnp.float32),
            pltpu.VMEM_SHARED((N_PAD, D), jnp.float32),
            pltpu.SemaphoreType.DMA((IBUF,)),
            pltpu.SemaphoreType.DMA((IBUF,)),
            pltpu.SemaphoreType.DMA((NBUF,)),
            pltpu.SemaphoreType.DMA,
        ],
    )
    def k(x_hbm, sd_hbm, attr_hbm, we_hbm, be_hbm, out_hbm,
          sd, av_ring, gbuf, we_v, be_v, accum, isem, asem, gsem, ssem):
        cid = lax.axis_index("c")
        tid = lax.axis_index("s")
        bpw = jnp.where(cid == 0, bpw0, bpw1)
        wbase = cid * (NS * bpw0) + tid * bpw

        def load_idx(i):
            s = lax.rem(i, IBUF)
            pltpu.async_copy(sd_hbm.at[wbase + i], sd.at[s], isem.at[s])
            pltpu.async_copy(attr_hbm.at[wbase + i], av_ring.at[s],
                             asem.at[s])

        load_idx(0)
        load_idx(1)
        pltpu.sync_copy(we_hbm, we_v)
        pltpu.sync_copy(be_hbm, be_v)

        zero16 = jnp.zeros((16,), jnp.float32)

        def zrow(i, _):
            for c in range(NCHUNK):
                gbuf[0, i, pl.ds(c * 16, 16)] = zero16
            return 0

        lax.fori_loop(0, OCHUNK, zrow, 0)
        for j in range(ROWS_PER_TILE // OCHUNK):
            pltpu.sync_copy(
                gbuf.at[0],
                accum.at[pl.ds(tid * ROWS_PER_TILE + j * OCHUNK, OCHUNK)])
        plsc.subcore_barrier()

        def load_idx_wait(i):
            s = lax.rem(i, IBUF)
            pltpu.make_async_copy(sd_hbm.at[wbase + i], sd.at[s],
                                  isem.at[s]).wait()
            pltpu.make_async_copy(attr_hbm.at[wbase + i], av_ring.at[s],
                                  asem.at[s]).wait()

        def gather(i):
            s, p = lax.rem(i, IBUF), lax.rem(i, NBUF)
            pltpu.async_copy(x_hbm.at[sd.at[s, 0]],
                             gbuf.at[p, pl.ds(0, B // 2)], gsem.at[p])
            pltpu.async_copy(x_hbm.at[sd.at[s, 1]],
                             gbuf.at[p, pl.ds(B // 2, B // 2)], gsem.at[p])

        def gather_wait(i):
            s, p = lax.rem(i, IBUF), lax.rem(i, NBUF)
            pltpu.make_async_copy(x_hbm.at[sd.at[s, 0]],
                                  gbuf.at[p, pl.ds(0, B // 2)],
                                  gsem.at[p]).wait()
            pltpu.make_async_copy(x_hbm.at[sd.at[s, 1]],
                                  gbuf.at[p, pl.ds(B // 2, B // 2)],
                                  gsem.at[p]).wait()

        def scatter_half(i, h):
            s, p = lax.rem(i, IBUF), lax.rem(i, NBUF)
            idx = plsc.Indices(sd.at[s, 2 + h], ignored_value=-1)
            pltpu.async_copy(gbuf.at[p, pl.ds(h * (B // 2), B // 2)],
                             accum.at[idx], ssem, add=True)

        def scatter_wait(i):
            s, p = lax.rem(i, IBUF), lax.rem(i, NBUF)
            for h in range(2):
                idx = plsc.Indices(sd.at[s, 2 + h], ignored_value=-1)
                pltpu.make_async_copy(
                    gbuf.at[p, pl.ds(h * (B // 2), B // 2)],
                    accum.at[idx], ssem).wait()

        def compute_half(i, h):
            s, p = lax.rem(i, IBUF), lax.rem(i, NBUF)
            gh = (B // 32) * h

            @plsc.parallel_loop(gh, gh + B // 32, 1, unroll=1)
            def group_body(g):
                av = [av_ring[s, kk, pl.ds(g * 16, 16)] for kk in range(DE)]
                for j in range(16):
                    e = g * 16 + j
                    a = [av[kk][j] for kk in range(DE)]
                    gvs = [gbuf[p, e, pl.ds(c * 16, 16)]
                           for c in range(NCHUNK)]
                    ms = []
                    for c in range(NCHUNK):
                        t1 = a[0] * wv[c][0] + a[1] * wv[c][1]
                        t2 = a[2] * wv[c][2] + a[3] * wv[c][3]
                        ms.append(jnp.maximum((gvs[c] + bv[c]) + (t1 + t2),
                                              0.0))
                    for c in range(NCHUNK):
                        gbuf[p, e, pl.ds(c * 16, 16)] = ms[c]

        wv = [[we_v[kk, pl.ds(c * 16, 16)] for kk in range(DE)]
              for c in range(NCHUNK)]
        bv = [be_v[pl.ds(c * 16, 16)] for c in range(NCHUNK)]

        load_idx_wait(0)
        gather(0)

        def step(i, _):
            @pl.when(i >= 1)
            def _():
                scatter_wait(i - 1)

            @pl.when(i + 2 < bpw)
            def _():
                load_idx(i + 2)

            @pl.when(i + 1 < bpw)
            def _():
                load_idx_wait(i + 1)
                gather(i + 1)

            gather_wait(i)
            compute_half(i, 0)
            scatter_half(i, 0)
            compute_half(i, 1)
            scatter_half(i, 1)
            return 0

        lax.fori_loop(0, bpw, step, 0)
        scatter_wait(bpw - 1)
        plsc.subcore_barrier()

        def owrite(go):
            for j in range(ROWS_PER_TILE // OCHUNK):
                off = tid * ROWS_PER_TILE + j * OCHUNK

                @pl.when(off + OCHUNK <= N)
                def _():
                    go(accum.at[pl.ds(off, OCHUNK)],
                       out_hbm.at[pl.ds(cid * N + off, OCHUNK)])

                @pl.when(off == (N // OCHUNK) * OCHUNK)
                def _():
                    t = N - (N // OCHUNK) * OCHUNK
                    go(accum.at[pl.ds(off, t)],
                       out_hbm.at[pl.ds(cid * N + off, t)])

        owrite(lambda a, b: pltpu.async_copy(a, b, gsem.at[0]))
        owrite(lambda a, b: pltpu.make_async_copy(a, b, gsem.at[0]).wait())

    return k


def _mlp_kernel(p0_ref, p1_ref, x_ref, w1_ref, b1_ref, w2_ref, b2_ref, o_ref):
    h = p0_ref[...] + p1_ref[...] + x_ref[...]
    h1 = jnp.maximum(
        jnp.dot(h, w1_ref[...], preferred_element_type=jnp.float32)
        + b1_ref[...], 0.0)
    o_ref[...] = (jnp.dot(h1, w2_ref[...], preferred_element_type=jnp.float32)
                  + b2_ref[...])


def kernel(x, edge_index, edge_attr, We, be, W1, b1, W2, b2):
    e = edge_index.shape[1]
    bpw_sum = -(-e // (NS * B))
    bpw0 = min(BPW0, bpw_sum)
    bpw1 = bpw_sum - bpw0
    e_pad = NS * bpw_sum * B
    src = edge_index[0].astype(jnp.int32)
    dst = edge_index[1].astype(jnp.int32)
    pad = e_pad - e
    src = jnp.pad(src, (0, pad))
    dst = jnp.pad(dst, (0, pad), constant_values=-1)
    attr = jnp.pad(edge_attr.astype(jnp.float32), ((0, pad), (0, 0)))
    nb = e_pad // B
    sd3 = jnp.concatenate(
        [src.reshape(nb, 2, B // 2), dst.reshape(nb, 2, B // 2)], axis=1)
    attr3 = attr.reshape(nb, B, DE).transpose(0, 2, 1)
    parts = _sc_edge_kernel(bpw0, bpw1)(x, sd3, attr3, We, be)

    blk = 1000
    out = pl.pallas_call(
        _mlp_kernel,
        grid=(N // blk,),
        in_specs=[
            pl.BlockSpec((blk, D), lambda i: (i, 0)),
            pl.BlockSpec((blk, D), lambda i: (N // blk + i, 0)),
            pl.BlockSpec((blk, D), lambda i: (i, 0)),
            pl.BlockSpec((D, D), lambda i: (0, 0)),
            pl.BlockSpec((1, D), lambda i: (0, 0)),
            pl.BlockSpec((D, D), lambda i: (0, 0)),
            pl.BlockSpec((1, D), lambda i: (0, 0)),
        ],
        out_specs=pl.BlockSpec((blk, D), lambda i: (i, 0)),
        out_shape=jax.ShapeDtypeStruct((N, D), jnp.float32),
    )(parts, parts, x, W1, b1.reshape(1, D), W2, b2.reshape(1, D))
    return out
